# baseline (speedup 1.0000x reference)
import functools

import jax
import jax.numpy as jnp
from jax import lax
from jax.experimental import pallas as pl
from jax.experimental.pallas import tpu as pltpu
from jax.experimental.pallas import tpu_sc as plsc

B = 4096
L = 200
H = 64
VOC = 1000000
NC = 2
NS = 16
NW = NC * NS
BPW = B // NW
LANES = 16
NBUF = 4
LEG = 128

VBLK = 128
NBLK_MAIN = 244
V_MAIN = NBLK_MAIN * NW * VBLK

_PICK_DNUMS = lax.GatherDimensionNumbers(
    offset_dims=(), collapsed_slice_dims=(0,), start_index_map=(0,)
)


def _bcast_lane(vec, lane_idx):
    return lax.gather(
        vec,
        lane_idx[:, None],
        _PICK_DNUMS,
        (1,),
        mode=lax.GatherScatterMode.PROMISE_IN_BOUNDS,
    )


def _mesh():
    return plsc.VectorSubcoreMesh(
        core_axis_name="c", subcore_axis_name="s", num_cores=NC, num_subcores=NS
    )


@functools.partial(
    pl.kernel,
    out_type=jax.ShapeDtypeStruct((VOC // 2, 2 * H), jnp.float32),
    mesh=_mesh(),
    compiler_params=pltpu.CompilerParams(
        needs_layout_passes=False, use_tc_tiling_on_sc=True
    ),
    scratch_types=[
        pltpu.VMEM((H, VBLK), jnp.float32),
        pltpu.VMEM((H, VBLK), jnp.float32),
        pltpu.VMEM((H, VBLK), jnp.float32),
        pltpu.VMEM((H, VBLK), jnp.float32),
        pltpu.SemaphoreType.DMA,
        pltpu.SemaphoreType.DMA,
        pltpu.SemaphoreType.DMA,
        pltpu.SemaphoreType.DMA,
    ],
)
def _repack(tab_t_hbm, tail_hbm, rep_hbm, xa, xb, ya, yb, sxa, sxb, sya, syb):
    w = lax.axis_index("s") * NC + lax.axis_index("c")

    def blk_v0(k):
        return pl.multiple_of((k * NW + w) * VBLK, VBLK)

    def fire_in(k, x, sem):
        pltpu.async_copy(tab_t_hbm.at[:, pl.ds(blk_v0(k), VBLK)], x, sem)

    def wait_in(k, x, sem):
        pltpu.make_async_copy(tab_t_hbm.at[:, pl.ds(blk_v0(k), VBLK)], x, sem).wait()

    def out_dst(k):
        return rep_hbm.at[pl.ds(pl.multiple_of(blk_v0(k) // 2, H), H)]

    rows_m = [
        (16 * (m % 4)) + lax.iota(jnp.int32, LANES) for m in range(8)
    ]
    zeros16 = jnp.zeros((LANES,), jnp.int32)

    def transpose(x, y, nr=H):
        def row_body(r, carry):
            col_e = zeros16 + 2 * r
            col_o = col_e + 1
            for m in range(8):
                col = col_e if m < 4 else col_o
                y[r, pl.ds(16 * m, 16)] = plsc.load_gather(x, [rows_m[m], col])
            return carry

        lax.fori_loop(0, nr, row_body, 0, unroll=1)

    fire_in(0, xa, sxa)
    fire_in(1, xb, sxb)

    def pair_body(i, carry):
        ke = 2 * i
        wait_in(ke, xa, sxa)

        @pl.when(i > 0)
        def _():
            pltpu.make_async_copy(ya, out_dst(ke - 2), sya).wait()

        transpose(xa, ya)

        @pl.when(i < (NBLK_MAIN // 2) - 1)
        def _():
            fire_in(ke + 2, xa, sxa)

        pltpu.async_copy(ya, out_dst(ke), sya)

        wait_in(ke + 1, xb, sxb)

        @pl.when(i > 0)
        def _():
            pltpu.make_async_copy(yb, out_dst(ke - 1), syb).wait()

        transpose(xb, yb)

        @pl.when(i < (NBLK_MAIN // 2) - 1)
        def _():
            fire_in(ke + 3, xb, sxb)

        pltpu.async_copy(yb, out_dst(ke + 1), syb)
        return carry

    lax.fori_loop(0, NBLK_MAIN // 2, pair_body, 0, unroll=1)
    pltpu.make_async_copy(ya, out_dst(NBLK_MAIN - 2), sya).wait()
    pltpu.make_async_copy(yb, out_dst(NBLK_MAIN - 1), syb).wait()

    @pl.when(w < 4)
    def _():
        v0t = pl.multiple_of(V_MAIN + w * VBLK, VBLK)
        pltpu.sync_copy(tab_t_hbm.at[:, pl.ds(v0t, VBLK)], xa)
        transpose(xa, ya)
        pltpu.sync_copy(ya, rep_hbm.at[pl.ds(pl.multiple_of(v0t // 2, H), H)])

    @pl.when(w == 4)
    def _():
        pltpu.sync_copy(tail_hbm, xa.at[pl.ds(0, H // 2)])
        pltpu.sync_copy(
            xa.at[pl.ds(0, H // 2)], rep_hbm.at[pl.ds((VOC - H) // 2, H // 2)]
        )


@functools.partial(
    pl.kernel,
    out_type=jax.ShapeDtypeStruct((B, L), jnp.float32),
    mesh=_mesh(),
    compiler_params=pltpu.CompilerParams(
        needs_layout_passes=False, use_tc_tiling_on_sc=False
    ),
    scratch_types=[
        pltpu.VMEM((BPW, H), jnp.float32),
        pltpu.VMEM((BPW, L), jnp.int32),
        pltpu.VMEM((NBUF, L, H), jnp.float32),
        pltpu.VMEM((NBUF, 208), jnp.float32),
        pltpu.SemaphoreType.DMA,
        pltpu.SemaphoreType.DMA,
    ],
)
def _entity_head(
    hid_hbm, kb_hbm, tab_hbm, out_hbm, hid_v, idx_v, rows_v, outb_v, gsem, osem
):
    wid = lax.axis_index("s") * NC + lax.axis_index("c")
    b0 = wid * BPW
    pltpu.sync_copy(hid_hbm.at[pl.ds(b0, BPW)], hid_v)
    pltpu.sync_copy(kb_hbm.at[pl.ds(b0, BPW)], idx_v)

    def fire_gather(bl, slot):
        pltpu.async_copy(
            tab_hbm.at[idx_v.at[bl, pl.ds(0, LEG)]],
            rows_v.at[slot, pl.ds(0, LEG)],
            gsem,
        )
        pltpu.async_copy(
            tab_hbm.at[idx_v.at[bl, pl.ds(LEG, L - LEG)]],
            rows_v.at[slot, pl.ds(LEG, L - LEG)],
            gsem,
        )

    for p in range(NBUF):
        fire_gather(p, p)

    def batch_body(bl, carry):
        slot = lax.rem(bl, NBUF)
        pltpu.make_async_copy(
            tab_hbm.at[idx_v.at[bl, pl.ds(0, LEG)]],
            rows_v.at[slot, pl.ds(0, LEG)],
            gsem,
        ).wait()
        pltpu.make_async_copy(
            tab_hbm.at[idx_v.at[bl, pl.ds(LEG, L - LEG)]],
            rows_v.at[slot, pl.ds(LEG, L - LEG)],
            gsem,
        ).wait()

        @pl.when(bl >= NBUF)
        def _():
            pltpu.make_async_copy(
                outb_v.at[slot, pl.ds(0, L)], out_hbm.at[b0 + bl - NBUF], osem
            ).wait()

        hv = [hid_v[bl, pl.ds(c * LANES, LANES)] for c in range(H // LANES)]
        lane_iota = lax.iota(jnp.int32, LANES)
        pick15 = jnp.full((LANES,), LANES - 1, jnp.int32)

        def dot16(t):
            prod = rows_v[slot, t, pl.ds(0, LANES)] * hv[0]
            for c in range(1, H // LANES):
                prod = prod + rows_v[slot, t, pl.ds(c * LANES, LANES)] * hv[c]
            csum = plsc.cumsum(prod)
            return _bcast_lane(csum, pick15)

        def blk(t0, n_tok):
            res = jnp.zeros((LANES,), jnp.float32)
            for k in range(n_tok):
                res = jnp.where(lane_iota == k, dot16(t0 + k), res)
            outb_v[slot, pl.ds(t0, LANES)] = res

        def blk_body(i, carry2):
            blk(i * LANES, LANES)
            return carry2

        lax.fori_loop(0, L // LANES, blk_body, 0, unroll=1)
        blk((L // LANES) * LANES, L - (L // LANES) * LANES)

        @pl.when(bl + NBUF < BPW)
        def _():
            fire_gather(bl + NBUF, slot)

        pltpu.async_copy(outb_v.at[slot, pl.ds(0, L)], out_hbm.at[b0 + bl], osem)
        return carry

    lax.fori_loop(0, BPW, batch_body, 0, unroll=1)

    for p in range(NBUF):
        bl = BPW - NBUF + p
        pltpu.make_async_copy(
            outb_v.at[lax.rem(jnp.int32(bl), NBUF), pl.ds(0, L)],
            out_hbm.at[b0 + bl],
            osem,
        ).wait()


def kernel(hidden_state, kb_arr, global_pointer, emb_table):
    del global_pointer
    kb = kb_arr.astype(jnp.int32)
    tail_packed = emb_table[VOC - H :].reshape(H // 2, 2 * H)
    rep = _repack(emb_table.T, tail_packed)
    rep_rows = rep.reshape(VOC, H)
    return _entity_head(hidden_state, kb, rep_rows)
